# baseline (speedup 1.0000x reference)
import functools

import jax
import jax.numpy as jnp
from jax import lax
from jax.experimental import pallas as pl
from jax.experimental.pallas import tpu as pltpu
from jax.experimental.pallas import tpu_sc as plsc

B = 16384
D = 128
H = 512

NC = 2
NS = 16
NW = NC * NS
BPW = B // NW
SUB = 128
NSUB = BPW // SUB


def _make_sc_gather(compute_gmf):
  mesh = plsc.VectorSubcoreMesh(core_axis_name="c", subcore_axis_name="s")

  if compute_gmf:
    out_type = [jax.ShapeDtypeStruct((NW, BPW), jnp.float32)]
  else:
    out_type = [jax.ShapeDtypeStruct((B, D), jnp.float32),
                jax.ShapeDtypeStruct((B, D), jnp.float32)]

  if compute_gmf:
    cost = pl.CostEstimate(flops=3 * B * D, transcendentals=0,
                           bytes_accessed=2 * B * D * 4 + B * 4)
  else:
    cost = pl.CostEstimate(flops=0, transcendentals=0,
                           bytes_accessed=4 * B * D * 4)

  @functools.partial(
      pl.kernel,
      mesh=mesh,
      out_type=out_type,
      cost_estimate=cost,
      scratch_types=[
          [pltpu.VMEM((SUB,), jnp.int32)] * NSUB,
          [pltpu.VMEM((SUB,), jnp.int32)] * NSUB,
          pltpu.VMEM((D,), jnp.float32),
          [pltpu.VMEM((SUB, D), jnp.float32)] * 2,
          [pltpu.VMEM((SUB, D), jnp.float32)] * 2,
          pltpu.VMEM((BPW,), jnp.float32),
          [pltpu.SemaphoreType.DMA] * 4,
          [pltpu.SemaphoreType.DMA] * 4,
          pltpu.SemaphoreType.DMA,
      ],
  )
  def sc_gather(uidx_hbm, midx_hbm, ta_hbm, tb_hbm, w3a_hbm,
                *outs_and_scratch):
    if compute_gmf:
      (gd_out, uidx_v, midx_v, w3a_v, a_b, b_b, gd_buf, gsem, osem,
       isem) = outs_and_scratch
      ao_out = bo_out = None
    else:
      (ao_out, bo_out, uidx_v, midx_v, w3a_v, a_b, b_b, gd_buf, gsem, osem,
       isem) = outs_and_scratch
      gd_out = None
    wid = lax.axis_index("s") * NC + lax.axis_index("c")
    base = wid * BPW
    icps = []
    for s in range(NSUB):
      icps.append(pltpu.async_copy(uidx_hbm.at[wid, s], uidx_v[s], isem))
      icps.append(pltpu.async_copy(midx_hbm.at[wid, s], midx_v[s], isem))
    if compute_gmf:
      pltpu.sync_copy(w3a_hbm, w3a_v)
    for cp in icps:
      cp.wait()

    def issue_gathers(s):
      k = s % 2
      return (pltpu.async_copy(ta_hbm.at[uidx_v[s]], a_b[k], gsem[2 * k]),
              pltpu.async_copy(tb_hbm.at[midx_v[s]], b_b[k],
                               gsem[2 * k + 1]))

    lane = lax.iota(jnp.int32, 16)

    def compute_gd(s):
      k = s % 2
      eu, em = a_b[k], b_b[k]

      def grp_body(g, carry):
        tot = jnp.zeros((16,), jnp.float32)
        for rr in range(16):
          r = g * 16 + rr
          p = [eu[r, pl.ds(c * 16, 16)] * em[r, pl.ds(c * 16, 16)]
               * w3a_v[pl.ds(c * 16, 16)] for c in range(D // 16)]
          acc = ((p[0] + p[1]) + (p[2] + p[3])) + ((p[4] + p[5])
                                                   + (p[6] + p[7]))
          for m in (1, 2, 4, 8):
            acc = acc + acc.at[lane ^ m].get(mode="promise_in_bounds")
          tot = jnp.where(lane == rr, acc, tot)
        gd_buf[pl.ds(s * SUB + g * 16, 16)] = tot
        return carry

      lax.fori_loop(0, SUB // 16, grp_body, 0)

    gathers = [None] * NSUB
    copyouts = [None] * NSUB
    gathers[0] = issue_gathers(0)
    for s in range(NSUB):
      k = s % 2
      if s + 1 < NSUB:
        if s >= 1 and not compute_gmf:
          for cp in copyouts[s - 1]:
            cp.wait()
        gathers[s + 1] = issue_gathers(s + 1)
      gathers[s][0].wait()
      gathers[s][1].wait()
      if compute_gmf:
        compute_gd(s)
      else:
        off = base + s * SUB
        copyouts[s] = (
            pltpu.async_copy(a_b[k], ao_out.at[pl.ds(off, SUB)],
                             osem[2 * k]),
            pltpu.async_copy(b_b[k], bo_out.at[pl.ds(off, SUB)],
                             osem[2 * k + 1]),
        )
    if compute_gmf:
      pltpu.sync_copy(gd_buf, gd_out.at[wid])
    else:
      for cp in copyouts[NSUB - 2] + copyouts[NSUB - 1]:
        cp.wait()

  return sc_gather


def _tc_dense_body(mu, mm, W1r, b1r, W2r, b2r, w3mr, b3r, out, wc, bc):
  i = pl.program_id(0)

  @pl.when(i == 0)
  def _():
    wcf = jnp.dot(W1r[...], W2r[...], preferred_element_type=jnp.float32)
    wc[...] = wcf.astype(jnp.bfloat16)
    bc[...] = (jnp.dot(b1r[...], W2r[...], preferred_element_type=jnp.float32)
               + b2r[...])

  h = (jnp.dot(mu[...].astype(jnp.bfloat16), wc[0:D, :],
               preferred_element_type=jnp.float32)
       + jnp.dot(mm[...].astype(jnp.bfloat16), wc[D:2 * D, :],
                 preferred_element_type=jnp.float32)
       + bc[...])
  hr = jnp.maximum(h, 0.0)
  out[...] = jnp.sum(hr * w3mr[...], axis=1) + b3r[0, 0]


def _tc_dense(mu, mm, W1, b1, W2, b2, W3, b3):
  bs = 2048
  grid = (B // bs,)
  row = lambda i: (i, 0)
  const = lambda i: (0, 0)
  return pl.pallas_call(
      _tc_dense_body,
      grid=grid,
      in_specs=[
          pl.BlockSpec((bs, D), row),
          pl.BlockSpec((bs, D), row),
          pl.BlockSpec((2 * D, H), const),
          pl.BlockSpec((1, H), const),
          pl.BlockSpec((H, 2 * D), const),
          pl.BlockSpec((1, 2 * D), const),
          pl.BlockSpec((1, 2 * D), const),
          pl.BlockSpec((1, 1), const),
      ],
      out_specs=pl.BlockSpec((bs,), lambda i: (i,)),
      out_shape=jax.ShapeDtypeStruct((B,), jnp.float32),
      scratch_shapes=[
          pltpu.VMEM((2 * D, 2 * D), jnp.bfloat16),
          pltpu.VMEM((1, 2 * D), jnp.float32),
      ],
      compiler_params=pltpu.CompilerParams(
          dimension_semantics=("arbitrary",)),
      cost_estimate=pl.CostEstimate(
          flops=2 * B * 2 * D * 2 * D, transcendentals=0,
          bytes_accessed=2 * B * D * 4),
  )(mu, mm, W1, b1.reshape(1, H), W2, b2.reshape(1, 2 * D),
    W3[D:, 0].reshape(1, 2 * D), b3.reshape(1, 1))


def kernel(x, gmf_user, gmf_movie, mlp_user, mlp_movie, W1, b1, W2, b2, W3,
           b3):
  user = x[:, 0].reshape(NW, NSUB, SUB)
  movie = x[:, 1].reshape(NW, NSUB, SUB)
  rating = x[:, 2]
  w3a = W3[:D, 0]
  sc_mlp = _make_sc_gather(compute_gmf=False)
  sc_gmf = _make_sc_gather(compute_gmf=True)
  mu, mm = sc_mlp(user, movie, mlp_user, mlp_movie, w3a)
  mlp_out = _tc_dense(mu, mm, W1, b1, W2, b2, W3, b3)
  (gd,) = sc_gmf(user, movie, gmf_user, gmf_movie, w3a)
  out = (mlp_out + gd.reshape(B)).reshape(B, 1)
  return out, rating


# speedup vs baseline: 1.1079x; 1.1079x over previous
import functools

import jax
import jax.numpy as jnp
from jax import lax
from jax.experimental import pallas as pl
from jax.experimental.pallas import tpu as pltpu
from jax.experimental.pallas import tpu_sc as plsc

B = 16384
D = 128
H = 512

NC = 2
NS = 16
NW = NC * NS
BPW = B // NW
SUB = 64
NSUB = BPW // SUB


def _make_sc_gather():
  mesh = plsc.VectorSubcoreMesh(core_axis_name="c", subcore_axis_name="s")

  @functools.partial(
      pl.kernel,
      mesh=mesh,
      out_type=[
          jax.ShapeDtypeStruct((B, D), jnp.float32),
          jax.ShapeDtypeStruct((B, D), jnp.float32),
          jax.ShapeDtypeStruct((NW, BPW), jnp.float32),
      ],
      cost_estimate=pl.CostEstimate(
          flops=3 * B * D, transcendentals=0,
          bytes_accessed=4 * B * D * 4 + 2 * B * D * 4),
      scratch_types=[
          [pltpu.VMEM((SUB,), jnp.int32)] * NSUB,
          [pltpu.VMEM((SUB,), jnp.int32)] * NSUB,
          pltpu.VMEM((D,), jnp.float32),
          [pltpu.VMEM((SUB, D), jnp.float32)] * 2,
          [pltpu.VMEM((SUB, D), jnp.float32)] * 2,
          [pltpu.VMEM((SUB, D), jnp.float32)] * 2,
          [pltpu.VMEM((SUB, D), jnp.float32)] * 2,
          pltpu.VMEM((BPW,), jnp.float32),
          [pltpu.SemaphoreType.DMA] * 8,
          [pltpu.SemaphoreType.DMA] * 4,
          pltpu.SemaphoreType.DMA,
      ],
  )
  def sc_gather(uidx_hbm, midx_hbm, gu_hbm, gm_hbm, mu_hbm, mm_hbm, w3a_hbm,
                muo_out, mmo_out, gd_out,
                uidx_v, midx_v, w3a_v, eu_b, em_b, mu_b, mm_b, gd_buf,
                gsem, osem, isem):
    wid = lax.axis_index("s") * NC + lax.axis_index("c")
    base = wid * BPW
    icps = []
    for s in range(NSUB):
      icps.append(pltpu.async_copy(uidx_hbm.at[wid, s], uidx_v[s], isem))
      icps.append(pltpu.async_copy(midx_hbm.at[wid, s], midx_v[s], isem))
    pltpu.sync_copy(w3a_hbm, w3a_v)
    for cp in icps:
      cp.wait()

    def issue_gathers(s):
      k = s % 2
      ui, mi = uidx_v[s], midx_v[s]
      return (pltpu.async_copy(gu_hbm.at[ui], eu_b[k], gsem[4 * k + 0]),
              pltpu.async_copy(gm_hbm.at[mi], em_b[k], gsem[4 * k + 1]),
              pltpu.async_copy(mu_hbm.at[ui], mu_b[k], gsem[4 * k + 2]),
              pltpu.async_copy(mm_hbm.at[mi], mm_b[k], gsem[4 * k + 3]))

    lane = lax.iota(jnp.int32, 16)

    def compute_gd(s):
      k = s % 2
      eu, em = eu_b[k], em_b[k]

      def grp_body(g, carry):
        tot = jnp.zeros((16,), jnp.float32)
        for rr in range(16):
          r = g * 16 + rr
          p = [eu[r, pl.ds(c * 16, 16)] * em[r, pl.ds(c * 16, 16)]
               * w3a_v[pl.ds(c * 16, 16)] for c in range(D // 16)]
          acc = ((p[0] + p[1]) + (p[2] + p[3])) + ((p[4] + p[5])
                                                   + (p[6] + p[7]))
          for m in (1, 2, 4, 8):
            acc = acc + acc.at[lane ^ m].get(mode="promise_in_bounds")
          tot = jnp.where(lane == rr, acc, tot)
        gd_buf[pl.ds(s * SUB + g * 16, 16)] = tot
        return carry

      lax.fori_loop(0, SUB // 16, grp_body, 0)

    gathers = [None] * NSUB
    copyouts = [None] * NSUB
    gathers[0] = issue_gathers(0)
    for s in range(NSUB):
      k = s % 2
      if s + 1 < NSUB:
        if s >= 1:
          for cp in copyouts[s - 1]:
            cp.wait()
        gathers[s + 1] = issue_gathers(s + 1)
      gathers[s][2].wait()
      gathers[s][3].wait()
      off = base + s * SUB
      copyouts[s] = (
          pltpu.async_copy(mu_b[k], muo_out.at[pl.ds(off, SUB)],
                           osem[2 * k + 0]),
          pltpu.async_copy(mm_b[k], mmo_out.at[pl.ds(off, SUB)],
                           osem[2 * k + 1]),
      )
      gathers[s][0].wait()
      gathers[s][1].wait()
      compute_gd(s)
    for cp in copyouts[NSUB - 2] + copyouts[NSUB - 1]:
      cp.wait()
    pltpu.sync_copy(gd_buf, gd_out.at[wid])

  return sc_gather


def _tc_fold_body(W1r, b1r, W2r, b2r, wc_out, bc_out):
  wcf = jnp.dot(W1r[...], W2r[...], preferred_element_type=jnp.float32)
  wc_out[...] = wcf.astype(jnp.bfloat16)
  bc_out[...] = (jnp.dot(b1r[...], W2r[...],
                         preferred_element_type=jnp.float32) + b2r[...])


def _tc_fold(W1, b1, W2, b2):
  return pl.pallas_call(
      _tc_fold_body,
      out_shape=[jax.ShapeDtypeStruct((2 * D, 2 * D), jnp.bfloat16),
                 jax.ShapeDtypeStruct((1, 2 * D), jnp.float32)],
  )(W1, b1.reshape(1, H), W2, b2.reshape(1, 2 * D))


def _tc_dense_body(mu, mm, wcr, bcr, w3mr, b3r, out):
  h = (jnp.dot(mu[...].astype(jnp.bfloat16), wcr[0:D, :],
               preferred_element_type=jnp.float32)
       + jnp.dot(mm[...].astype(jnp.bfloat16), wcr[D:2 * D, :],
                 preferred_element_type=jnp.float32)
       + bcr[...])
  hr = jnp.maximum(h, 0.0)
  out[...] = jnp.sum(hr * w3mr[...], axis=1) + b3r[0, 0]


def _tc_dense(mu, mm, wc, bc, W3, b3):
  bs = 2048
  grid = (B // bs,)
  row = lambda i: (i, 0)
  const = lambda i: (0, 0)
  return pl.pallas_call(
      _tc_dense_body,
      grid=grid,
      in_specs=[
          pl.BlockSpec((bs, D), row),
          pl.BlockSpec((bs, D), row),
          pl.BlockSpec((2 * D, 2 * D), const),
          pl.BlockSpec((1, 2 * D), const),
          pl.BlockSpec((1, 2 * D), const),
          pl.BlockSpec((1, 1), const),
      ],
      out_specs=pl.BlockSpec((bs,), lambda i: (i,)),
      out_shape=jax.ShapeDtypeStruct((B,), jnp.float32),
      compiler_params=pltpu.CompilerParams(
          dimension_semantics=("parallel",)),
      cost_estimate=pl.CostEstimate(
          flops=2 * B * 2 * D * 2 * D, transcendentals=0,
          bytes_accessed=2 * B * D * 4),
  )(mu, mm, wc, bc, W3[D:, 0].reshape(1, 2 * D), b3.reshape(1, 1))


def kernel(x, gmf_user, gmf_movie, mlp_user, mlp_movie, W1, b1, W2, b2, W3,
           b3):
  user = x[:, 0].reshape(NW, NSUB, SUB)
  movie = x[:, 1].reshape(NW, NSUB, SUB)
  rating = x[:, 2]
  w3a = W3[:D, 0]
  sc_gather = _make_sc_gather()
  mu, mm, gd = sc_gather(user, movie, gmf_user, gmf_movie, mlp_user,
                         mlp_movie, w3a)
  wc, bc = _tc_fold(W1, b1, W2, b2)
  mlp_out = _tc_dense(mu, mm, wc, bc, W3, b3)
  out = (mlp_out + gd.reshape(B)).reshape(B, 1)
  return out, rating


# speedup vs baseline: 1.3284x; 1.1990x over previous
import functools

import jax
import jax.numpy as jnp
from jax import lax
from jax.experimental import pallas as pl
from jax.experimental.pallas import tpu as pltpu
from jax.experimental.pallas import tpu_sc as plsc

B = 16384
D = 128
H = 512

NC = 2
NS = 16
NW = NC * NS
BPW = B // NW
SUB = 64
NSUB = BPW // SUB


def _make_sc_gather():
  mesh = plsc.VectorSubcoreMesh(core_axis_name="c", subcore_axis_name="s")

  @functools.partial(
      pl.kernel,
      mesh=mesh,
      out_type=[
          jax.ShapeDtypeStruct((B, D), jnp.float32),
          jax.ShapeDtypeStruct((B, D), jnp.float32),
          jax.ShapeDtypeStruct((NW, BPW), jnp.float32),
      ],
      cost_estimate=pl.CostEstimate(
          flops=3 * B * D, transcendentals=0,
          bytes_accessed=4 * B * D * 4 + 2 * B * D * 4),
      scratch_types=[
          [pltpu.VMEM((SUB,), jnp.int32)] * NSUB,
          [pltpu.VMEM((SUB,), jnp.int32)] * NSUB,
          pltpu.VMEM((D,), jnp.float32),
          [pltpu.VMEM((SUB, D), jnp.float32)] * 2,
          [pltpu.VMEM((SUB, D), jnp.float32)] * 2,
          [pltpu.VMEM((SUB, D), jnp.float32)] * 2,
          [pltpu.VMEM((SUB, D), jnp.float32)] * 2,
          pltpu.VMEM((BPW,), jnp.float32),
          [pltpu.SemaphoreType.DMA] * 8,
          [pltpu.SemaphoreType.DMA] * 4,
          pltpu.SemaphoreType.DMA,
      ],
  )
  def sc_gather(uidx_hbm, midx_hbm, gu_hbm, gm_hbm, mu_hbm, mm_hbm, w3a_hbm,
                muo_out, mmo_out, gd_out,
                uidx_v, midx_v, w3a_v, eu_b, em_b, mu_b, mm_b, gd_buf,
                gsem, osem, isem):
    wid = lax.axis_index("s") * NC + lax.axis_index("c")
    base = wid * BPW
    icps = []
    for s in range(NSUB):
      icps.append(pltpu.async_copy(uidx_hbm.at[wid, s], uidx_v[s], isem))
      icps.append(pltpu.async_copy(midx_hbm.at[wid, s], midx_v[s], isem))
    pltpu.sync_copy(w3a_hbm, w3a_v)
    for cp in icps:
      cp.wait()

    def issue_gathers(s):
      k = s % 2
      ui, mi = uidx_v[s], midx_v[s]
      return (pltpu.async_copy(gu_hbm.at[ui], eu_b[k], gsem[4 * k + 0]),
              pltpu.async_copy(gm_hbm.at[mi], em_b[k], gsem[4 * k + 1]),
              pltpu.async_copy(mu_hbm.at[ui], mu_b[k], gsem[4 * k + 2]),
              pltpu.async_copy(mm_hbm.at[mi], mm_b[k], gsem[4 * k + 3]))

    lane = lax.iota(jnp.int32, 16)

    def compute_gd(s):
      k = s % 2
      eu, em = eu_b[k], em_b[k]

      def grp_body(g, carry):
        tot = jnp.zeros((16,), jnp.float32)
        for rr in range(16):
          r = g * 16 + rr
          p = [eu[r, pl.ds(c * 16, 16)] * em[r, pl.ds(c * 16, 16)]
               * w3a_v[pl.ds(c * 16, 16)] for c in range(D // 16)]
          acc = ((p[0] + p[1]) + (p[2] + p[3])) + ((p[4] + p[5])
                                                   + (p[6] + p[7]))
          for m in (1, 2, 4, 8):
            acc = acc + acc.at[lane ^ m].get(mode="promise_in_bounds")
          tot = jnp.where(lane == rr, acc, tot)
        gd_buf[pl.ds(s * SUB + g * 16, 16)] = tot
        return carry

      lax.fori_loop(0, SUB // 16, grp_body, 0)

    gathers = [None] * NSUB
    copyouts = [None] * NSUB
    gathers[0] = issue_gathers(0)
    for s in range(NSUB):
      k = s % 2
      if s + 1 < NSUB:
        if s >= 1:
          for cp in copyouts[s - 1]:
            cp.wait()
        gathers[s + 1] = issue_gathers(s + 1)
      gathers[s][2].wait()
      gathers[s][3].wait()
      off = base + s * SUB
      copyouts[s] = (
          pltpu.async_copy(mu_b[k], muo_out.at[pl.ds(off, SUB)],
                           osem[2 * k + 0]),
          pltpu.async_copy(mm_b[k], mmo_out.at[pl.ds(off, SUB)],
                           osem[2 * k + 1]),
      )
      gathers[s][0].wait()
      gathers[s][1].wait()
      compute_gd(s)
    for cp in copyouts[NSUB - 2] + copyouts[NSUB - 1]:
      cp.wait()
    pltpu.sync_copy(gd_buf, gd_out.at[wid])

  return sc_gather


def _tc_fold_body(W1r, b1r, W2r, b2r, wc_out, bc_out):
  wc_out[...] = jnp.dot(W1r[...], W2r[...],
                        preferred_element_type=jnp.float32)
  bc_out[...] = (jnp.dot(b1r[...], W2r[...],
                         preferred_element_type=jnp.float32) + b2r[...])


def _tc_fold(W1, b1, W2, b2):
  return pl.pallas_call(
      _tc_fold_body,
      out_shape=[jax.ShapeDtypeStruct((2 * D, 2 * D), jnp.float32),
                 jax.ShapeDtypeStruct((1, 2 * D), jnp.float32)],
  )(W1, b1.reshape(1, H), W2, b2.reshape(1, 2 * D))


def _tc_dense_body(mu, mm, wcr, bcr, w3mr, b3r, out):
  fast = jax.lax.Precision.DEFAULT
  h = (jnp.dot(mu[...], wcr[0:D, :], precision=fast,
               preferred_element_type=jnp.float32)
       + jnp.dot(mm[...], wcr[D:2 * D, :], precision=fast,
                 preferred_element_type=jnp.float32)
       + bcr[...])
  hr = jnp.maximum(h, 0.0)
  o2 = jnp.dot(hr, w3mr[...], precision=fast,
               preferred_element_type=jnp.float32)
  out[...] = o2[:, 0] + b3r[0, 0]


def _tc_dense(mu, mm, wc, bc, W3, b3):
  bs = 2048
  grid = (B // bs,)
  row = lambda i: (i, 0)
  const = lambda i: (0, 0)
  return pl.pallas_call(
      _tc_dense_body,
      grid=grid,
      in_specs=[
          pl.BlockSpec((bs, D), row),
          pl.BlockSpec((bs, D), row),
          pl.BlockSpec((2 * D, 2 * D), const),
          pl.BlockSpec((1, 2 * D), const),
          pl.BlockSpec((2 * D, 1), const),
          pl.BlockSpec((1, 1), const),
      ],
      out_specs=pl.BlockSpec((bs,), lambda i: (i,)),
      out_shape=jax.ShapeDtypeStruct((B,), jnp.float32),
      compiler_params=pltpu.CompilerParams(
          dimension_semantics=("parallel",)),
      cost_estimate=pl.CostEstimate(
          flops=2 * B * 2 * D * 2 * D, transcendentals=0,
          bytes_accessed=2 * B * D * 4),
  )(mu, mm, wc, bc, W3[D:, :], b3.reshape(1, 1))


def kernel(x, gmf_user, gmf_movie, mlp_user, mlp_movie, W1, b1, W2, b2, W3,
           b3):
  user = x[:, 0].reshape(NW, NSUB, SUB)
  movie = x[:, 1].reshape(NW, NSUB, SUB)
  rating = x[:, 2]
  w3a = W3[:D, 0]
  sc_gather = _make_sc_gather()
  mu, mm, gd = sc_gather(user, movie, gmf_user, gmf_movie, mlp_user,
                         mlp_movie, w3a)
  wc, bc = _tc_fold(W1, b1, W2, b2)
  mlp_out = _tc_dense(mu, mm, wc, bc, W3, b3)
  out = (mlp_out + gd.reshape(B)).reshape(B, 1)
  return out, rating


# speedup vs baseline: 1.3528x; 1.0184x over previous
import functools

import jax
import jax.numpy as jnp
from jax import lax
from jax.experimental import pallas as pl
from jax.experimental.pallas import tpu as pltpu
from jax.experimental.pallas import tpu_sc as plsc

B = 16384
D = 128
H = 512

NC = 2
NS = 16
NW = NC * NS
BPW = B // NW
SUB = 64
NSUB = BPW // SUB
SLOTS = 3


def _make_sc_gather():
  mesh = plsc.VectorSubcoreMesh(core_axis_name="c", subcore_axis_name="s")

  @functools.partial(
      pl.kernel,
      mesh=mesh,
      out_type=[
          jax.ShapeDtypeStruct((B, D), jnp.float32),
          jax.ShapeDtypeStruct((B, D), jnp.float32),
          jax.ShapeDtypeStruct((NW, BPW), jnp.float32),
      ],
      cost_estimate=pl.CostEstimate(
          flops=3 * B * D, transcendentals=0,
          bytes_accessed=4 * B * D * 4 + 2 * B * D * 4),
      scratch_types=[
          [pltpu.VMEM((SUB,), jnp.int32)] * NSUB,
          [pltpu.VMEM((SUB,), jnp.int32)] * NSUB,
          pltpu.VMEM((D,), jnp.float32),
          [pltpu.VMEM((SUB, D), jnp.float32)] * SLOTS,
          [pltpu.VMEM((SUB, D), jnp.float32)] * SLOTS,
          [pltpu.VMEM((SUB, D), jnp.float32)] * SLOTS,
          [pltpu.VMEM((SUB, D), jnp.float32)] * SLOTS,
          pltpu.VMEM((BPW,), jnp.float32),
          [pltpu.SemaphoreType.DMA] * (4 * SLOTS),
          [pltpu.SemaphoreType.DMA] * (2 * SLOTS),
          pltpu.SemaphoreType.DMA,
      ],
  )
  def sc_gather(uidx_hbm, midx_hbm, gu_hbm, gm_hbm, mu_hbm, mm_hbm, w3a_hbm,
                muo_out, mmo_out, gd_out,
                uidx_v, midx_v, w3a_v, eu_b, em_b, mu_b, mm_b, gd_buf,
                gsem, osem, isem):
    wid = lax.axis_index("s") * NC + lax.axis_index("c")
    base = wid * BPW
    icps = []
    for s in range(NSUB):
      icps.append(pltpu.async_copy(uidx_hbm.at[wid, s], uidx_v[s], isem))
      icps.append(pltpu.async_copy(midx_hbm.at[wid, s], midx_v[s], isem))
    pltpu.sync_copy(w3a_hbm, w3a_v)
    for cp in icps:
      cp.wait()

    def issue_gathers(s):
      k = s % SLOTS
      ui, mi = uidx_v[s], midx_v[s]
      return (pltpu.async_copy(gu_hbm.at[ui], eu_b[k], gsem[4 * k + 0]),
              pltpu.async_copy(gm_hbm.at[mi], em_b[k], gsem[4 * k + 1]),
              pltpu.async_copy(mu_hbm.at[ui], mu_b[k], gsem[4 * k + 2]),
              pltpu.async_copy(mm_hbm.at[mi], mm_b[k], gsem[4 * k + 3]))

    lane = lax.iota(jnp.int32, 16)

    def compute_gd(s):
      k = s % SLOTS
      eu, em = eu_b[k], em_b[k]

      def grp_body(g, carry):
        tot = jnp.zeros((16,), jnp.float32)
        for rr in range(16):
          r = g * 16 + rr
          p = [eu[r, pl.ds(c * 16, 16)] * em[r, pl.ds(c * 16, 16)]
               * w3a_v[pl.ds(c * 16, 16)] for c in range(D // 16)]
          acc = ((p[0] + p[1]) + (p[2] + p[3])) + ((p[4] + p[5])
                                                   + (p[6] + p[7]))
          for m in (1, 2, 4, 8):
            acc = acc + acc.at[lane ^ m].get(mode="promise_in_bounds")
          tot = jnp.where(lane == rr, acc, tot)
        gd_buf[pl.ds(s * SUB + g * 16, 16)] = tot
        return carry

      lax.fori_loop(0, SUB // 16, grp_body, 0)

    gathers = [None] * NSUB
    copyouts = [None] * NSUB
    gathers[0] = issue_gathers(0)
    gathers[1] = issue_gathers(1)
    for s in range(NSUB):
      k = s % SLOTS
      if s + 2 < NSUB:
        if s + 2 >= SLOTS:
          for cp in copyouts[s + 2 - SLOTS]:
            cp.wait()
        gathers[s + 2] = issue_gathers(s + 2)
      gathers[s][2].wait()
      gathers[s][3].wait()
      off = base + s * SUB
      copyouts[s] = (
          pltpu.async_copy(mu_b[k], muo_out.at[pl.ds(off, SUB)],
                           osem[2 * k + 0]),
          pltpu.async_copy(mm_b[k], mmo_out.at[pl.ds(off, SUB)],
                           osem[2 * k + 1]),
      )
      gathers[s][0].wait()
      gathers[s][1].wait()
      compute_gd(s)
    for s in range(max(0, NSUB - SLOTS), NSUB):
      for cp in copyouts[s]:
        cp.wait()
    pltpu.sync_copy(gd_buf, gd_out.at[wid])

  return sc_gather


def _tc_fold_body(W1r, b1r, W2r, b2r, wc_out, bc_out):
  wc_out[...] = jnp.dot(W1r[...], W2r[...],
                        preferred_element_type=jnp.float32)
  bc_out[...] = (jnp.dot(b1r[...], W2r[...],
                         preferred_element_type=jnp.float32) + b2r[...])


def _tc_fold(W1, b1, W2, b2):
  return pl.pallas_call(
      _tc_fold_body,
      out_shape=[jax.ShapeDtypeStruct((2 * D, 2 * D), jnp.float32),
                 jax.ShapeDtypeStruct((1, 2 * D), jnp.float32)],
  )(W1, b1.reshape(1, H), W2, b2.reshape(1, 2 * D))


def _tc_dense_body(mu, mm, wcr, bcr, w3mr, b3r, out):
  fast = jax.lax.Precision.DEFAULT
  h = (jnp.dot(mu[...], wcr[0:D, :], precision=fast,
               preferred_element_type=jnp.float32)
       + jnp.dot(mm[...], wcr[D:2 * D, :], precision=fast,
                 preferred_element_type=jnp.float32)
       + bcr[...])
  hr = jnp.maximum(h, 0.0)
  o2 = jnp.dot(hr, w3mr[...], precision=fast,
               preferred_element_type=jnp.float32)
  out[...] = o2[:, 0] + b3r[0, 0]


def _tc_dense(mu, mm, wc, bc, W3, b3):
  bs = 4096
  grid = (B // bs,)
  row = lambda i: (i, 0)
  const = lambda i: (0, 0)
  return pl.pallas_call(
      _tc_dense_body,
      grid=grid,
      in_specs=[
          pl.BlockSpec((bs, D), row),
          pl.BlockSpec((bs, D), row),
          pl.BlockSpec((2 * D, 2 * D), const),
          pl.BlockSpec((1, 2 * D), const),
          pl.BlockSpec((2 * D, 1), const),
          pl.BlockSpec((1, 1), const),
      ],
      out_specs=pl.BlockSpec((bs,), lambda i: (i,)),
      out_shape=jax.ShapeDtypeStruct((B,), jnp.float32),
      compiler_params=pltpu.CompilerParams(
          dimension_semantics=("parallel",)),
      cost_estimate=pl.CostEstimate(
          flops=2 * B * 2 * D * 2 * D, transcendentals=0,
          bytes_accessed=2 * B * D * 4),
  )(mu, mm, wc, bc, W3[D:, :], b3.reshape(1, 1))


def kernel(x, gmf_user, gmf_movie, mlp_user, mlp_movie, W1, b1, W2, b2, W3,
           b3):
  user = x[:, 0].reshape(NW, NSUB, SUB)
  movie = x[:, 1].reshape(NW, NSUB, SUB)
  rating = x[:, 2]
  w3a = W3[:D, 0]
  sc_gather = _make_sc_gather()
  mu, mm, gd = sc_gather(user, movie, gmf_user, gmf_movie, mlp_user,
                         mlp_movie, w3a)
  wc, bc = _tc_fold(W1, b1, W2, b2)
  mlp_out = _tc_dense(mu, mm, wc, bc, W3, b3)
  out = (mlp_out + gd.reshape(B)).reshape(B, 1)
  return out, rating
